# baseline (speedup 1.0000x reference)
import functools

import jax
import jax.numpy as jnp
from jax import lax
from jax.experimental import pallas as pl
from jax.experimental.pallas import tpu as pltpu
from jax.experimental.pallas import tpu_sc as plsc

N = 10000
E = 320000
D_IN = 128
S = 64
C = 40
BOUND = 10.0

NC = 2
NS = 16
F = S // NC
K = 128
EPT = -(-E // NS)
BLK = 512
NBLK = -(-EPT // BLK)
EPT_PAD = NBLK * BLK
SRC_LEN = (NBLK + 1) * BLK
NPAD = -(-N // (NS * 8)) * NS * 8
SLAB = NPAD // NS
HSLAB = N // NS


def _tc_in_body(x_ref, w_ref, b_ref, o_ref):
    r = (jnp.dot(x_ref[...], w_ref[...], preferred_element_type=jnp.float32)
         + b_ref[...])
    o_ref[0:N, :] = r[:, 0:F]
    o_ref[N:2 * N, :] = r[:, F:S]


def _tc_in(x, W_in, b_in):
    return pl.pallas_call(
        _tc_in_body,
        out_shape=jax.ShapeDtypeStruct((NC * N, F), jnp.float32),
    )(x, W_in, b_in.reshape(1, S))


def _tc_update_body(h_ref, parts_ref, w_ref, b_ref, o_ref):
    h = jnp.concatenate([h_ref[0:N, :], h_ref[N:2 * N, :]], axis=1)
    agg = jnp.concatenate([parts_ref[0:N, :], parts_ref[NPAD:NPAD + N, :]],
                          axis=1)
    clamped = jnp.clip(agg, 0.0, BOUND)
    r = (jnp.dot(h, w_ref[0:S, :], preferred_element_type=jnp.float32)
         + jnp.dot(clamped, w_ref[S:2 * S, :],
                   preferred_element_type=jnp.float32)
         + b_ref[...])
    o_ref[0:N, :] = r[:, 0:F]
    o_ref[N:2 * N, :] = r[:, F:S]


def _tc_update(h, parts, W, b):
    return pl.pallas_call(
        _tc_update_body,
        out_shape=jax.ShapeDtypeStruct((NC * N, F), jnp.float32),
    )(h, parts, W, b.reshape(1, S))


def _tc_finish_body(h_ref, parts_ref, w_ref, b_ref, wo_ref, bo_ref, o_ref):
    h = jnp.concatenate([h_ref[0:N, :], h_ref[N:2 * N, :]], axis=1)
    agg = jnp.concatenate([parts_ref[0:N, :], parts_ref[NPAD:NPAD + N, :]],
                          axis=1)
    clamped = jnp.clip(agg, 0.0, BOUND)
    h2 = (jnp.dot(h, w_ref[0:S, :], preferred_element_type=jnp.float32)
          + jnp.dot(clamped, w_ref[S:2 * S, :],
                    preferred_element_type=jnp.float32)
          + b_ref[...])
    logits = (jnp.dot(h2, wo_ref[...], preferred_element_type=jnp.float32)
              + bo_ref[...])
    m = jnp.max(logits, axis=-1, keepdims=True)
    z = logits - m
    lse = jnp.log(jnp.sum(jnp.exp(z), axis=-1, keepdims=True))
    o_ref[...] = z - lse


def _tc_finish(h, parts, W, b, W_out, b_out):
    return pl.pallas_call(
        _tc_finish_body,
        out_shape=jax.ShapeDtypeStruct((N, C), jnp.float32),
    )(h, parts, W, b.reshape(1, S), W_out, b_out.reshape(1, C))


def _sc_body(h_hbm, src_hbm, dst_hbm, zeros_hbm, out_hbm,
             src_v, dst_v, rows_v, h_sh, agg_sh, gsem, ssem):
    c = lax.axis_index("c")
    s = lax.axis_index("s")

    d0 = pltpu.async_copy(zeros_hbm, agg_sh.at[pl.ds(s * SLAB, SLAB)],
                          ssem.at[0])
    d1 = pltpu.async_copy(h_hbm.at[pl.ds(c * N + s * HSLAB, HSLAB)],
                          h_sh.at[pl.ds(s * HSLAB, HSLAB)], ssem.at[1])
    d2 = pltpu.async_copy(src_hbm.at[pl.ds(s * EPT, EPT)],
                          src_v.at[pl.ds(0, EPT)], gsem.at[0])
    d3 = pltpu.async_copy(dst_hbm.at[pl.ds(s * EPT, EPT)],
                          dst_v.at[pl.ds(0, EPT)], gsem.at[1])
    d2.wait()
    d3.wait()
    zi = jnp.zeros((16,), jnp.int32)
    for i in range(EPT, SRC_LEN, 16):
        src_v[pl.ds(i, 16)] = zi
    ni = jnp.full((16,), N, jnp.int32)
    for i in range(EPT, NBLK * BLK, 16):
        dst_v[pl.ds(i, 16)] = ni
    d0.wait()
    d1.wait()
    plsc.subcore_barrier()

    pltpu.async_copy(h_sh.at[src_v.at[pl.ds(0, BLK)]], rows_v.at[0],
                     gsem.at[0])

    def chunk(j, carry):
        b = j % 2
        pltpu.make_async_copy(h_sh.at[src_v.at[pl.ds(j * BLK, BLK)]],
                              rows_v.at[b], gsem.at[b]).wait()
        pltpu.async_copy(h_sh.at[src_v.at[pl.ds((j + 1) * BLK, BLK)]],
                         rows_v.at[1 - b], gsem.at[1 - b])
        pltpu.sync_copy(rows_v.at[b],
                        agg_sh.at[dst_v.at[pl.ds(j * BLK, BLK)]],
                        add=True)
        return carry

    lax.fori_loop(0, NBLK, chunk, 0, unroll=False)
    lb = NBLK % 2
    pltpu.make_async_copy(h_sh.at[src_v.at[pl.ds(0, BLK)]],
                          rows_v.at[lb], gsem.at[lb]).wait()
    plsc.subcore_barrier()
    pltpu.sync_copy(agg_sh.at[pl.ds(s * SLAB, SLAB)],
                    out_hbm.at[pl.ds(c * NPAD + s * SLAB, SLAB)])


@functools.partial(
    pl.kernel,
    out_type=jax.ShapeDtypeStruct((NC * NPAD, F), jnp.float32),
    mesh=plsc.VectorSubcoreMesh(core_axis_name="c", subcore_axis_name="s",
                                num_cores=NC, num_subcores=NS),
    compiler_params=pltpu.CompilerParams(use_tc_tiling_on_sc=False),
    scratch_types=[
        pltpu.VMEM((SRC_LEN,), jnp.int32),
        pltpu.VMEM((NBLK * BLK,), jnp.int32),
        pltpu.VMEM((2, BLK, F), jnp.float32),
        pltpu.VMEM_SHARED((N, F), jnp.float32),
        pltpu.VMEM_SHARED((NPAD, F), jnp.float32),
        pltpu.SemaphoreType.DMA((2,)),
        pltpu.SemaphoreType.DMA((2,)),
    ],
)
def _sc_layer(*args):
    _sc_body(*args)


def kernel(x, edge_index, W_in, b_in, W0, b0, W1, b1, W_out, b_out):
    src = edge_index[0].astype(jnp.int32)
    dst = edge_index[1].astype(jnp.int32)
    zeros = jnp.zeros((SLAB, F), jnp.float32)

    h = _tc_in(x.astype(jnp.float32), W_in, b_in)
    parts = _sc_layer(h, src, dst, zeros)
    h = _tc_update(h, parts, W0, b0)
    parts = _sc_layer(h, src, dst, zeros)
    return _tc_finish(h, parts, W1, b1, W_out, b_out)


# speedup vs baseline: 1.0437x; 1.0437x over previous
import functools

import jax
import jax.numpy as jnp
from jax import lax
from jax.experimental import pallas as pl
from jax.experimental.pallas import tpu as pltpu
from jax.experimental.pallas import tpu_sc as plsc

N = 10000
E = 320000
D_IN = 128
S = 64
C = 40
BOUND = 10.0

NC = 2
NS = 16
F = S // NC
K = 128
EPT = -(-E // NS)
BLK = 512
NBLK = -(-EPT // BLK)
EPT_PAD = NBLK * BLK
SRC_LEN = (NBLK + 1) * BLK
NPAD = -(-N // (NS * 8)) * NS * 8
SLAB = NPAD // NS
HSLAB = N // NS


def _tc_in_body(x_ref, w_ref, b_ref, o_ref):
    r = (jnp.dot(x_ref[...], w_ref[...], preferred_element_type=jnp.float32)
         + b_ref[...])
    o_ref[0:N, :] = r[:, 0:F]
    o_ref[N:2 * N, :] = r[:, F:S]


def _tc_in(x, W_in, b_in):
    return pl.pallas_call(
        _tc_in_body,
        out_shape=jax.ShapeDtypeStruct((NC * N, F), jnp.float32),
    )(x, W_in, b_in.reshape(1, S))


def _tc_update_body(h_ref, parts_ref, w_ref, b_ref, o_ref):
    h = jnp.concatenate([h_ref[0:N, :], h_ref[N:2 * N, :]], axis=1)
    agg = jnp.concatenate([parts_ref[0:N, :], parts_ref[NPAD:NPAD + N, :]],
                          axis=1)
    clamped = jnp.clip(agg, 0.0, BOUND)
    r = (jnp.dot(h, w_ref[0:S, :], preferred_element_type=jnp.float32)
         + jnp.dot(clamped, w_ref[S:2 * S, :],
                   preferred_element_type=jnp.float32)
         + b_ref[...])
    o_ref[0:N, :] = r[:, 0:F]
    o_ref[N:2 * N, :] = r[:, F:S]


def _tc_update(h, parts, W, b):
    return pl.pallas_call(
        _tc_update_body,
        out_shape=jax.ShapeDtypeStruct((NC * N, F), jnp.float32),
    )(h, parts, W, b.reshape(1, S))


def _tc_finish_body(h_ref, parts_ref, w_ref, b_ref, wo_ref, bo_ref, o_ref):
    h = jnp.concatenate([h_ref[0:N, :], h_ref[N:2 * N, :]], axis=1)
    agg = jnp.concatenate([parts_ref[0:N, :], parts_ref[NPAD:NPAD + N, :]],
                          axis=1)
    clamped = jnp.clip(agg, 0.0, BOUND)
    h2 = (jnp.dot(h, w_ref[0:S, :], preferred_element_type=jnp.float32)
          + jnp.dot(clamped, w_ref[S:2 * S, :],
                    preferred_element_type=jnp.float32)
          + b_ref[...])
    logits = (jnp.dot(h2, wo_ref[...], preferred_element_type=jnp.float32)
              + bo_ref[...])
    m = jnp.max(logits, axis=-1, keepdims=True)
    z = logits - m
    lse = jnp.log(jnp.sum(jnp.exp(z), axis=-1, keepdims=True))
    o_ref[...] = z - lse


def _tc_finish(h, parts, W, b, W_out, b_out):
    return pl.pallas_call(
        _tc_finish_body,
        out_shape=jax.ShapeDtypeStruct((N, C), jnp.float32),
    )(h, parts, W, b.reshape(1, S), W_out, b_out.reshape(1, C))


def _sc_body(h_hbm, src_hbm, dst_hbm, zeros_hbm, out_hbm,
             src_v, dst_v, rows_v, h_sh, agg_sh, gsem, ssem):
    c = lax.axis_index("c")
    s = lax.axis_index("s")

    d0 = pltpu.async_copy(zeros_hbm, agg_sh.at[pl.ds(s * SLAB, SLAB)],
                          ssem.at[0])
    d1 = pltpu.async_copy(h_hbm.at[pl.ds(c * N + s * HSLAB, HSLAB)],
                          h_sh.at[pl.ds(s * HSLAB, HSLAB)], ssem.at[1])
    d2 = pltpu.async_copy(src_hbm.at[s], src_v, gsem.at[0])
    d3 = pltpu.async_copy(dst_hbm.at[s], dst_v, gsem.at[1])
    d0.wait()
    d1.wait()
    d2.wait()
    d3.wait()
    plsc.subcore_barrier()

    pltpu.async_copy(h_sh.at[src_v.at[pl.ds(0, BLK)]], rows_v.at[0],
                     gsem.at[0])

    def chunk(j, carry):
        b = j % 2
        pltpu.make_async_copy(h_sh.at[src_v.at[pl.ds(j * BLK, BLK)]],
                              rows_v.at[b], gsem.at[b]).wait()
        pltpu.async_copy(h_sh.at[src_v.at[pl.ds((j + 1) * BLK, BLK)]],
                         rows_v.at[1 - b], gsem.at[1 - b])
        pltpu.sync_copy(rows_v.at[b],
                        agg_sh.at[dst_v.at[pl.ds(j * BLK, BLK)]],
                        add=True)
        return carry

    lax.fori_loop(0, NBLK, chunk, 0, unroll=False)
    lb = NBLK % 2
    pltpu.make_async_copy(h_sh.at[src_v.at[pl.ds(0, BLK)]],
                          rows_v.at[lb], gsem.at[lb]).wait()
    plsc.subcore_barrier()
    pltpu.sync_copy(agg_sh.at[pl.ds(s * SLAB, SLAB)],
                    out_hbm.at[pl.ds(c * NPAD + s * SLAB, SLAB)])


@functools.partial(
    pl.kernel,
    out_type=jax.ShapeDtypeStruct((NC * NPAD, F), jnp.float32),
    mesh=plsc.VectorSubcoreMesh(core_axis_name="c", subcore_axis_name="s",
                                num_cores=NC, num_subcores=NS),
    compiler_params=pltpu.CompilerParams(use_tc_tiling_on_sc=False),
    scratch_types=[
        pltpu.VMEM((SRC_LEN,), jnp.int32),
        pltpu.VMEM((NBLK * BLK,), jnp.int32),
        pltpu.VMEM((2, BLK, F), jnp.float32),
        pltpu.VMEM_SHARED((N, F), jnp.float32),
        pltpu.VMEM_SHARED((NPAD, F), jnp.float32),
        pltpu.SemaphoreType.DMA((2,)),
        pltpu.SemaphoreType.DMA((2,)),
    ],
)
def _sc_layer(*args):
    _sc_body(*args)


def kernel(x, edge_index, W_in, b_in, W0, b0, W1, b1, W_out, b_out):
    src = edge_index[0].astype(jnp.int32)
    dst = edge_index[1].astype(jnp.int32)
    pad = NS * EPT_PAD - E
    src3 = jnp.concatenate([src, jnp.zeros((pad,), jnp.int32)])
    src3 = src3.reshape(NS, EPT_PAD)
    src3 = jnp.concatenate([src3, jnp.zeros((NS, BLK), jnp.int32)], axis=1)
    dst3 = jnp.concatenate([dst, jnp.full((pad,), N, jnp.int32)])
    dst3 = dst3.reshape(NS, EPT_PAD)
    zeros = jnp.zeros((SLAB, F), jnp.float32)

    h = _tc_in(x.astype(jnp.float32), W_in, b_in)
    parts = _sc_layer(h, src3, dst3, zeros)
    h = _tc_update(h, parts, W0, b0)
    parts = _sc_layer(h, src3, dst3, zeros)
    return _tc_finish(h, parts, W1, b1, W_out, b_out)


# speedup vs baseline: 1.0749x; 1.0299x over previous
import functools

import jax
import jax.numpy as jnp
from jax import lax
from jax.experimental import pallas as pl
from jax.experimental.pallas import tpu as pltpu
from jax.experimental.pallas import tpu_sc as plsc

N = 10000
E = 320000
D_IN = 128
S = 64
C = 40
BOUND = 10.0

NC = 2
NS = 16
F = S // NC
K = 128
EPT = -(-E // NS)
BLK = 512
NBLK = -(-EPT // BLK)
EPT_PAD = NBLK * BLK
SRC_LEN = (NBLK + 1) * BLK
NPAD = -(-N // (NS * 8)) * NS * 8
SLAB = NPAD // NS
HSLAB = N // NS


def _tc_in_body(x_ref, w_ref, b_ref, o_ref):
    r = (jnp.dot(x_ref[...], w_ref[...], preferred_element_type=jnp.float32)
         + b_ref[...])
    o_ref[0:N, :] = r[:, 0:F]
    o_ref[N:2 * N, :] = r[:, F:S]


def _tc_in(x, W_in, b_in):
    return pl.pallas_call(
        _tc_in_body,
        out_shape=jax.ShapeDtypeStruct((NC * N, F), jnp.float32),
    )(x, W_in, b_in.reshape(1, S))


def _tc_update_body(h_ref, parts_ref, w_ref, b_ref, o_ref):
    h = jnp.concatenate([h_ref[0:N, :], h_ref[N:2 * N, :]], axis=1)
    agg = jnp.concatenate([parts_ref[0:N, :], parts_ref[NPAD:NPAD + N, :]],
                          axis=1)
    clamped = jnp.clip(agg, 0.0, BOUND)
    r = (jnp.dot(h, w_ref[0:S, :], preferred_element_type=jnp.float32)
         + jnp.dot(clamped, w_ref[S:2 * S, :],
                   preferred_element_type=jnp.float32)
         + b_ref[...])
    o_ref[0:N, :] = r[:, 0:F]
    o_ref[N:2 * N, :] = r[:, F:S]


def _tc_update(h, parts, W, b):
    return pl.pallas_call(
        _tc_update_body,
        out_shape=jax.ShapeDtypeStruct((NC * N, F), jnp.float32),
    )(h, parts, W, b.reshape(1, S))


def _tc_finish_body(h_ref, parts_ref, w_ref, b_ref, wo_ref, bo_ref, o_ref):
    h = jnp.concatenate([h_ref[0:N, :], h_ref[N:2 * N, :]], axis=1)
    agg = jnp.concatenate([parts_ref[0:N, :], parts_ref[NPAD:NPAD + N, :]],
                          axis=1)
    clamped = jnp.clip(agg, 0.0, BOUND)
    h2 = (jnp.dot(h, w_ref[0:S, :], preferred_element_type=jnp.float32)
          + jnp.dot(clamped, w_ref[S:2 * S, :],
                    preferred_element_type=jnp.float32)
          + b_ref[...])
    logits = (jnp.dot(h2, wo_ref[...], preferred_element_type=jnp.float32)
              + bo_ref[...])
    m = jnp.max(logits, axis=-1, keepdims=True)
    z = logits - m
    lse = jnp.log(jnp.sum(jnp.exp(z), axis=-1, keepdims=True))
    o_ref[...] = z - lse


def _tc_finish(h, parts, W, b, W_out, b_out):
    return pl.pallas_call(
        _tc_finish_body,
        out_shape=jax.ShapeDtypeStruct((N, C), jnp.float32),
    )(h, parts, W, b.reshape(1, S), W_out, b_out.reshape(1, C))


def _sc_body(h_hbm, src_hbm, dst_hbm, zeros_hbm, out_hbm,
             src_v, dst_v, rows_v, h_sh, agg_sh, gsem, ssem):
    c = lax.axis_index("c")
    s = lax.axis_index("s")

    d0 = pltpu.async_copy(zeros_hbm, agg_sh.at[pl.ds(s * SLAB, SLAB)],
                          ssem.at[0])
    d1 = pltpu.async_copy(h_hbm.at[pl.ds(c * N + s * HSLAB, HSLAB)],
                          h_sh.at[pl.ds(s * HSLAB, HSLAB)], ssem.at[1])
    d2 = pltpu.async_copy(src_hbm.at[s], src_v, gsem.at[0])
    d3 = pltpu.async_copy(dst_hbm.at[s], dst_v, gsem.at[1])
    d0.wait()
    d1.wait()
    d2.wait()
    d3.wait()
    plsc.subcore_barrier()

    pltpu.async_copy(h_sh.at[src_v.at[pl.ds(0, BLK)]], rows_v.at[0],
                     gsem.at[0])

    def chunk(j, carry):
        b = j % 3
        nb = (j + 1) % 3
        pltpu.make_async_copy(h_sh.at[src_v.at[pl.ds(j * BLK, BLK)]],
                              rows_v.at[b], gsem.at[b]).wait()

        @pl.when(j >= 2)
        def _():
            pltpu.make_async_copy(rows_v.at[nb],
                                  agg_sh.at[dst_v.at[pl.ds(0, BLK)]],
                                  ssem.at[nb]).wait()

        pltpu.async_copy(h_sh.at[src_v.at[pl.ds((j + 1) * BLK, BLK)]],
                         rows_v.at[nb], gsem.at[nb])
        pltpu.async_copy(rows_v.at[b],
                         agg_sh.at[dst_v.at[pl.ds(j * BLK, BLK)]],
                         ssem.at[b], add=True)
        return carry

    lax.fori_loop(0, NBLK, chunk, 0, unroll=False)
    for t in (NBLK - 2, NBLK - 1):
        pltpu.make_async_copy(rows_v.at[t % 3],
                              agg_sh.at[dst_v.at[pl.ds(0, BLK)]],
                              ssem.at[t % 3]).wait()
    pltpu.make_async_copy(h_sh.at[src_v.at[pl.ds(0, BLK)]],
                          rows_v.at[NBLK % 3], gsem.at[NBLK % 3]).wait()
    plsc.subcore_barrier()
    pltpu.sync_copy(agg_sh.at[pl.ds(s * SLAB, SLAB)],
                    out_hbm.at[pl.ds(c * NPAD + s * SLAB, SLAB)])


@functools.partial(
    pl.kernel,
    out_type=jax.ShapeDtypeStruct((NC * NPAD, F), jnp.float32),
    mesh=plsc.VectorSubcoreMesh(core_axis_name="c", subcore_axis_name="s",
                                num_cores=NC, num_subcores=NS),
    compiler_params=pltpu.CompilerParams(use_tc_tiling_on_sc=False),
    scratch_types=[
        pltpu.VMEM((SRC_LEN,), jnp.int32),
        pltpu.VMEM((NBLK * BLK,), jnp.int32),
        pltpu.VMEM((3, BLK, F), jnp.float32),
        pltpu.VMEM_SHARED((N, F), jnp.float32),
        pltpu.VMEM_SHARED((NPAD, F), jnp.float32),
        pltpu.SemaphoreType.DMA((3,)),
        pltpu.SemaphoreType.DMA((3,)),
    ],
)
def _sc_layer(*args):
    _sc_body(*args)


def kernel(x, edge_index, W_in, b_in, W0, b0, W1, b1, W_out, b_out):
    src = edge_index[0].astype(jnp.int32)
    dst = edge_index[1].astype(jnp.int32)
    pad = NS * EPT_PAD - E
    src3 = jnp.concatenate([src, jnp.zeros((pad,), jnp.int32)])
    src3 = src3.reshape(NS, EPT_PAD)
    src3 = jnp.concatenate([src3, jnp.zeros((NS, BLK), jnp.int32)], axis=1)
    dst3 = jnp.concatenate([dst, jnp.full((pad,), N, jnp.int32)])
    dst3 = dst3.reshape(NS, EPT_PAD)
    zeros = jnp.zeros((SLAB, F), jnp.float32)

    h = _tc_in(x.astype(jnp.float32), W_in, b_in)
    parts = _sc_layer(h, src3, dst3, zeros)
    h = _tc_update(h, parts, W0, b0)
    parts = _sc_layer(h, src3, dst3, zeros)
    return _tc_finish(h, parts, W1, b1, W_out, b_out)


# speedup vs baseline: 1.0773x; 1.0023x over previous
import functools

import jax
import jax.numpy as jnp
from jax import lax
from jax.experimental import pallas as pl
from jax.experimental.pallas import tpu as pltpu
from jax.experimental.pallas import tpu_sc as plsc

N = 10000
E = 320000
D_IN = 128
S = 64
C = 40
BOUND = 10.0

NC = 2
NS = 16
F = S // NC
K = 128
EPT = -(-E // NS)
BLK = 512
NBLK = -(-EPT // BLK)
EPT_PAD = NBLK * BLK
SRC_LEN = (NBLK + 1) * BLK
NPAD = -(-N // (NS * 8)) * NS * 8
SLAB = NPAD // NS
HSLAB = N // NS


def _tc_in_body(x_ref, w_ref, b_ref, o_ref):
    r = (jnp.dot(x_ref[...], w_ref[...], preferred_element_type=jnp.float32)
         + b_ref[...])
    o_ref[0:N, :] = r[:, 0:F]
    o_ref[N:2 * N, :] = r[:, F:S]


def _tc_in(x, W_in, b_in):
    return pl.pallas_call(
        _tc_in_body,
        out_shape=jax.ShapeDtypeStruct((NC * N, F), jnp.float32),
    )(x, W_in, b_in.reshape(1, S))


def _tc_update_body(h_ref, parts_ref, w_ref, b_ref, o_ref):
    h = jnp.concatenate([h_ref[0:N, :], h_ref[N:2 * N, :]], axis=1)
    agg = jnp.concatenate([parts_ref[0:N, :], parts_ref[NPAD:NPAD + N, :]],
                          axis=1)
    clamped = jnp.clip(agg, 0.0, BOUND)
    r = (jnp.dot(h, w_ref[0:S, :], preferred_element_type=jnp.float32)
         + jnp.dot(clamped, w_ref[S:2 * S, :],
                   preferred_element_type=jnp.float32)
         + b_ref[...])
    o_ref[0:N, :] = r[:, 0:F]
    o_ref[N:2 * N, :] = r[:, F:S]


def _tc_update(h, parts, W, b):
    return pl.pallas_call(
        _tc_update_body,
        out_shape=jax.ShapeDtypeStruct((NC * N, F), jnp.float32),
    )(h, parts, W, b.reshape(1, S))


def _tc_finish_body(h_ref, parts_ref, w_ref, b_ref, wo_ref, bo_ref, o_ref):
    h = jnp.concatenate([h_ref[0:N, :], h_ref[N:2 * N, :]], axis=1)
    agg = jnp.concatenate([parts_ref[0:N, :], parts_ref[NPAD:NPAD + N, :]],
                          axis=1)
    clamped = jnp.clip(agg, 0.0, BOUND)
    h2 = (jnp.dot(h, w_ref[0:S, :], preferred_element_type=jnp.float32)
          + jnp.dot(clamped, w_ref[S:2 * S, :],
                    preferred_element_type=jnp.float32)
          + b_ref[...])
    logits = (jnp.dot(h2, wo_ref[...], preferred_element_type=jnp.float32)
              + bo_ref[...])
    m = jnp.max(logits, axis=-1, keepdims=True)
    z = logits - m
    lse = jnp.log(jnp.sum(jnp.exp(z), axis=-1, keepdims=True))
    o_ref[...] = z - lse


def _tc_finish(h, parts, W, b, W_out, b_out):
    return pl.pallas_call(
        _tc_finish_body,
        out_shape=jax.ShapeDtypeStruct((N, C), jnp.float32),
    )(h, parts, W, b.reshape(1, S), W_out, b_out.reshape(1, C))


def _sc_body(h_hbm, src_hbm, dst_hbm, zeros_hbm, out_hbm,
             src_v, dst_v, rows_v, h_sh, agg_sh, gsem, ssem):
    c = lax.axis_index("c")
    s = lax.axis_index("s")

    d0 = pltpu.async_copy(zeros_hbm, agg_sh.at[pl.ds(s * SLAB, SLAB)],
                          ssem.at[0])
    d1 = pltpu.async_copy(h_hbm.at[pl.ds(c * N + s * HSLAB, HSLAB)],
                          h_sh.at[pl.ds(s * HSLAB, HSLAB)], ssem.at[1])
    d2 = pltpu.async_copy(src_hbm.at[s], src_v, gsem.at[0])
    d3 = pltpu.async_copy(dst_hbm.at[s], dst_v, gsem.at[1])
    d0.wait()
    d1.wait()
    d2.wait()
    d3.wait()
    plsc.subcore_barrier()

    pltpu.async_copy(h_sh.at[src_v.at[pl.ds(0, BLK)]], rows_v.at[0],
                     gsem.at[0])

    def chunk(j, carry):
        b = j % 3
        nb = (j + 1) % 3
        pltpu.make_async_copy(h_sh.at[src_v.at[pl.ds(j * BLK, BLK)]],
                              rows_v.at[b], gsem.at[b]).wait()

        @pl.when(j >= 2)
        def _():
            pltpu.make_async_copy(rows_v.at[nb],
                                  agg_sh.at[dst_v.at[pl.ds(0, BLK)]],
                                  ssem.at[nb]).wait()

        pltpu.async_copy(h_sh.at[src_v.at[pl.ds((j + 1) * BLK, BLK)]],
                         rows_v.at[nb], gsem.at[nb])
        pltpu.async_copy(rows_v.at[b],
                         agg_sh.at[dst_v.at[pl.ds(j * BLK, BLK)]],
                         ssem.at[b], add=True)
        return carry

    lax.fori_loop(0, NBLK, chunk, 0, unroll=False)
    for t in (NBLK - 2, NBLK - 1):
        pltpu.make_async_copy(rows_v.at[t % 3],
                              agg_sh.at[dst_v.at[pl.ds(0, BLK)]],
                              ssem.at[t % 3]).wait()
    pltpu.make_async_copy(h_sh.at[src_v.at[pl.ds(0, BLK)]],
                          rows_v.at[NBLK % 3], gsem.at[NBLK % 3]).wait()
    plsc.subcore_barrier()
    pltpu.sync_copy(agg_sh.at[pl.ds(s * SLAB, SLAB)],
                    out_hbm.at[pl.ds(c * NPAD + s * SLAB, SLAB)])


@functools.partial(
    pl.kernel,
    out_type=jax.ShapeDtypeStruct((NC * NPAD, F), jnp.float32),
    mesh=plsc.VectorSubcoreMesh(core_axis_name="c", subcore_axis_name="s",
                                num_cores=NC, num_subcores=NS),
    compiler_params=pltpu.CompilerParams(use_tc_tiling_on_sc=False,
                                         skip_device_barrier=True,
                                         disable_semaphore_checks=True),
    scratch_types=[
        pltpu.VMEM((SRC_LEN,), jnp.int32),
        pltpu.VMEM((NBLK * BLK,), jnp.int32),
        pltpu.VMEM((3, BLK, F), jnp.float32),
        pltpu.VMEM_SHARED((N, F), jnp.float32),
        pltpu.VMEM_SHARED((NPAD, F), jnp.float32),
        pltpu.SemaphoreType.DMA((3,)),
        pltpu.SemaphoreType.DMA((3,)),
    ],
)
def _sc_layer(*args):
    _sc_body(*args)


def kernel(x, edge_index, W_in, b_in, W0, b0, W1, b1, W_out, b_out):
    src = edge_index[0].astype(jnp.int32)
    dst = edge_index[1].astype(jnp.int32)
    pad = NS * EPT_PAD - E
    src3 = jnp.concatenate([src, jnp.zeros((pad,), jnp.int32)])
    src3 = src3.reshape(NS, EPT_PAD)
    src3 = jnp.concatenate([src3, jnp.zeros((NS, BLK), jnp.int32)], axis=1)
    dst3 = jnp.concatenate([dst, jnp.full((pad,), N, jnp.int32)])
    dst3 = dst3.reshape(NS, EPT_PAD)
    zeros = jnp.zeros((SLAB, F), jnp.float32)

    h = _tc_in(x.astype(jnp.float32), W_in, b_in)
    parts = _sc_layer(h, src3, dst3, zeros)
    h = _tc_update(h, parts, W0, b0)
    parts = _sc_layer(h, src3, dst3, zeros)
    return _tc_finish(h, parts, W1, b1, W_out, b_out)
